# Initial kernel scaffold; baseline (speedup 1.0000x reference)
import jax
import jax.numpy as jnp
from jax.experimental import pallas as pl


def kernel(x, edge_index, edge_attr, W1, b1, W2, b2, Wm1, bm1, Wm2, bm2):
    raise NotImplementedError("write your pallas kernel here")


# speedup vs baseline: 12.0962x; 12.0962x over previous
import functools

import jax
import jax.numpy as jnp
from jax import lax
from jax.experimental import pallas as pl
from jax.experimental.pallas import tpu as pltpu
from jax.experimental.pallas import tpu_sc as plsc

NC = 2
NS = 16
NW = NC * NS


def _mesh():
    return plsc.VectorSubcoreMesh(
        core_axis_name="c", subcore_axis_name="s",
        num_cores=NC, num_subcores=NS)


def _wid():
    return lax.axis_index("s") * NC + lax.axis_index("c")


def _split(total, nworkers, w):
    base, rem = total // nworkers, total % nworkers
    cnt = base + (w < rem).astype(jnp.int32)
    lo = w * base + jnp.minimum(w, rem)
    return lo, cnt


def _zero_spmem_slice(acc, zbuf, rows_per_tile, sid):
    zrows = zbuf.shape[0]
    zvec = jnp.zeros((16,), jnp.float32)

    def zstore(i, _):
        zbuf[i, :] = zvec
        return 0
    lax.fori_loop(0, zrows, zstore, 0)
    nrep = rows_per_tile // zrows

    def zcopy(k, _):
        pltpu.sync_copy(zbuf, acc.at[pl.ds(sid * rows_per_tile + k * zrows,
                                           zrows)])
        return 0
    lax.fori_loop(0, nrep, zcopy, 0)


def _make_deg_kernel(n, nrows):
    rpt = n // NS

    @functools.partial(
        pl.kernel,
        out_type=jax.ShapeDtypeStruct((NC * n, 16), jnp.float32),
        mesh=_mesh(),
        compiler_params=pltpu.CompilerParams(use_tc_tiling_on_sc=False, needs_layout_passes=False),
        scratch_types=[
            pltpu.VMEM((1, 128), jnp.int32),
            pltpu.VMEM((128, 16), jnp.float32),
            pltpu.VMEM((625, 16), jnp.float32),
            pltpu.VMEM_SHARED((n, 16), jnp.float32),
        ],
    )
    def deg_kernel(col_hbm, out_hbm, idx_v, ones_v, zbuf, acc):
        cid = lax.axis_index("c")
        sid = lax.axis_index("s")
        wid = _wid()
        one = jnp.ones((16,), jnp.float32)

        def ostore(i, _):
            ones_v[i, :] = one
            return 0
        lax.fori_loop(0, 128, ostore, 0)
        _zero_spmem_slice(acc, zbuf, rpt, sid)
        plsc.subcore_barrier()

        lo, cnt = _split(nrows, NW, wid)

        def body(j, _):
            pltpu.sync_copy(col_hbm.at[pl.ds(lo + j, 1)], idx_v)
            pltpu.sync_copy(ones_v, acc.at[idx_v.at[0]], add=True)
            return 0
        lax.fori_loop(0, cnt, body, 0)
        plsc.subcore_barrier()
        pltpu.sync_copy(acc.at[pl.ds(sid * rpt, rpt)],
                        out_hbm.at[pl.ds(cid * n + sid * rpt, rpt)])

    return deg_kernel


def _make_agg1_kernel(n, nrows):
    rpt = n // NS

    @functools.partial(
        pl.kernel,
        out_type=jax.ShapeDtypeStruct((NC * n, 16), jnp.float32),
        mesh=_mesh(),
        compiler_params=pltpu.CompilerParams(use_tc_tiling_on_sc=False, needs_layout_passes=False),
        scratch_types=[
            pltpu.VMEM((1, 128), jnp.int32),
            pltpu.VMEM((1, 128), jnp.int32),
            pltpu.VMEM((128, 16), jnp.float32),
            pltpu.VMEM((625, 16), jnp.float32),
            pltpu.VMEM_SHARED((n, 16), jnp.float32),
            pltpu.SemaphoreType.DMA,
        ],
    )
    def agg1_kernel(row_hbm, col_hbm, x16_hbm, out_hbm,
                    idxr, idxc, rows_v, zbuf, acc, sem):
        cid = lax.axis_index("c")
        sid = lax.axis_index("s")
        wid = _wid()
        _zero_spmem_slice(acc, zbuf, rpt, sid)
        plsc.subcore_barrier()

        lo, cnt = _split(nrows, NW, wid)

        def body(j, _):
            pltpu.sync_copy(row_hbm.at[pl.ds(lo + j, 1)], idxr)
            pltpu.sync_copy(col_hbm.at[pl.ds(lo + j, 1)], idxc)
            pltpu.async_copy(x16_hbm.at[idxr.at[0]], rows_v, sem).wait()
            pltpu.sync_copy(rows_v, acc.at[idxc.at[0]], add=True)
            return 0
        lax.fori_loop(0, cnt, body, 0)
        plsc.subcore_barrier()
        pltpu.sync_copy(acc.at[pl.ds(sid * rpt, rpt)],
                        out_hbm.at[pl.ds(cid * n + sid * rpt, rpt)])

    return agg1_kernel


def _make_agg2_kernel(n, nrows):
    rpt = n // NS

    @functools.partial(
        pl.kernel,
        out_type=jax.ShapeDtypeStruct((NC * n, 16), jnp.float32),
        mesh=_mesh(),
        compiler_params=pltpu.CompilerParams(use_tc_tiling_on_sc=False, needs_layout_passes=False),
        scratch_types=[
            pltpu.VMEM((1, 128), jnp.int32),
            pltpu.VMEM((1, 128), jnp.int32),
            pltpu.VMEM((128, 16), jnp.float32),
            pltpu.VMEM((625, 16), jnp.float32),
            pltpu.VMEM_SHARED((n, 16), jnp.float32),
            pltpu.SemaphoreType.DMA,
        ],
    )
    def agg2_kernel(row_hbm, col_hbm, zsplit_hbm, out_hbm,
                    idxr, idxc, rows_v, zbuf, acc, sem):
        cid = lax.axis_index("c")
        sid = lax.axis_index("s")
        _zero_spmem_slice(acc, zbuf, rpt, sid)
        plsc.subcore_barrier()

        lo, cnt = _split(nrows, NS, sid)
        offs = jnp.full((16,), 0, jnp.int32) + (cid * n).astype(jnp.int32)

        def body(j, _):
            pltpu.sync_copy(row_hbm.at[pl.ds(lo + j, 1)], idxr)
            pltpu.sync_copy(col_hbm.at[pl.ds(lo + j, 1)], idxc)
            for k in range(8):
                idxr[0, pl.ds(k * 16, 16)] = idxr[0, pl.ds(k * 16, 16)] + offs
            pltpu.async_copy(zsplit_hbm.at[idxr.at[0]], rows_v, sem).wait()
            pltpu.sync_copy(rows_v, acc.at[idxc.at[0]], add=True)
            return 0
        lax.fori_loop(0, cnt, body, 0)
        plsc.subcore_barrier()
        pltpu.sync_copy(acc.at[pl.ds(sid * rpt, rpt)],
                        out_hbm.at[pl.ds(cid * n + sid * rpt, rpt)])

    return agg2_kernel


def _make_mlp_kernel(n, nrows):
    @functools.partial(
        pl.kernel,
        out_type=jax.ShapeDtypeStruct((nrows, 128), jnp.float32),
        mesh=_mesh(),
        compiler_params=pltpu.CompilerParams(use_tc_tiling_on_sc=False, needs_layout_passes=False),
        scratch_types=[
            pltpu.VMEM((1, 128), jnp.int32),
            pltpu.VMEM((1, 128), jnp.int32),
            pltpu.VMEM((1, 128), jnp.float32),
            pltpu.VMEM((128, 16), jnp.float32),
            pltpu.VMEM((128, 16), jnp.float32),
            pltpu.VMEM((1, 128), jnp.float32),
            pltpu.VMEM((16, 16), jnp.float32),
            pltpu.VMEM((16, 16), jnp.float32),
            pltpu.VMEM((16,), jnp.float32),
            pltpu.SemaphoreType.DMA,
            pltpu.SemaphoreType.DMA,
        ],
    )
    def mlp_kernel(row_hbm, col_hbm, attr_hbm, pq_hbm, cmat_hbm, w2mat_hbm,
                   bm2_hbm, out_hbm,
                   idxr, idxc, attr_v, prow, qrow, outb,
                   cmat, w2mat, bm2v, semp, semq):
        wid = _wid()
        pltpu.sync_copy(cmat_hbm, cmat)
        pltpu.sync_copy(w2mat_hbm, w2mat)
        pltpu.sync_copy(bm2_hbm, bm2v)
        cj = [cmat[j, :] for j in range(16)]
        w2j = [w2mat[j, :] for j in range(16)]
        bias = bm2v[...]
        iota = lax.iota(jnp.int32, 16)
        nsplat = jnp.full((16,), n, jnp.int32)
        jsplat = [jnp.full((16,), j, jnp.int32) for j in range(16)]

        lo, cnt = _split(nrows, NW, wid)

        def body(j, _):
            pltpu.sync_copy(row_hbm.at[pl.ds(lo + j, 1)], idxr)
            pltpu.sync_copy(col_hbm.at[pl.ds(lo + j, 1)], idxc)
            pltpu.sync_copy(attr_hbm.at[pl.ds(lo + j, 1)], attr_v)
            for k in range(8):
                idxc[0, pl.ds(k * 16, 16)] = (idxc[0, pl.ds(k * 16, 16)]
                                              + nsplat)
            cp = pltpu.async_copy(pq_hbm.at[idxr.at[0]], prow, semp)
            cq = pltpu.async_copy(pq_hbm.at[idxc.at[0]], qrow, semq)
            cp.wait()
            cq.wait()

            def group(t, _):
                av = attr_v[0, pl.ds(t * 16, 16)]
                eids = jnp.full((16,), t * 16, jnp.int32) + iota
                acc = bias
                for jj in range(16):
                    pT = plsc.load_gather(prow, [eids, jsplat[jj]])
                    qT = plsc.load_gather(qrow, [eids, jsplat[jj]])
                    h = jnp.maximum(pT + qT + av * cj[jj], 0.0)
                    acc = acc + h * w2j[jj]
                y = 1.0 / (1.0 + jnp.exp(-acc))
                outb[0, pl.ds(t * 16, 16)] = y
                return 0
            lax.fori_loop(0, 8, group, 0)
            pltpu.sync_copy(outb, out_hbm.at[pl.ds(lo + j, 1)])
            return 0
        lax.fori_loop(0, cnt, body, 0)

    return mlp_kernel


def _tc_dis_kernel(ca_ref, cb_ref, x_ref, dis_ref, x16_ref):
    dis = lax.rsqrt(ca_ref[...] + cb_ref[...] + 1.0)
    dis_ref[...] = dis
    b = x_ref.shape[0]
    xpad = jnp.concatenate(
        [x_ref[...], jnp.zeros((b, 14), jnp.float32)], axis=1)
    x16_ref[...] = dis * xpad


def _tc_layer1_kernel(a0_ref, a1_ref, dis_ref, x_ref, w1_ref, b1_ref,
                      w2_ref, z_ref, zs_ref):
    dis2 = dis_ref[:, :2]
    s1 = dis2 * (a0_ref[:, :2] + a1_ref[:, :2]) + dis2 * dis2 * x_ref[...]
    h1 = (s1[:, 0:1] * w1_ref[0:1, :] + s1[:, 1:2] * w1_ref[1:2, :]
          + b1_ref[...])
    h1 = jnp.maximum(h1, 0.0)
    z = jnp.dot(h1, w2_ref[...], preferred_element_type=jnp.float32)
    z_ref[...] = z
    dis = dis_ref[...]
    zs_ref[...] = jnp.stack([dis * z[:, :16], dis * z[:, 16:]], axis=0)


def _tc_layer2_kernel(a0_ref, a1_ref, dis_ref, z_ref, b2_ref, wa_ref,
                      wb_ref, bm1_ref, pq_ref):
    dis = dis_ref[...]
    s2 = (jnp.concatenate([dis * a0_ref[...], dis * a1_ref[...]], axis=1)
          + jnp.concatenate([dis * dis, dis * dis], axis=1) * z_ref[...]
          + b2_ref[...])
    h2 = jnp.maximum(s2, 0.0)
    p = jnp.dot(h2, wa_ref[...], preferred_element_type=jnp.float32) \
        + bm1_ref[...]
    q = jnp.dot(h2, wb_ref[...], preferred_element_type=jnp.float32)
    pq_ref[...] = jnp.stack([p, q], axis=0)


def kernel(x, edge_index, edge_attr, W1, b1, W2, b2, Wm1, bm1, Wm2, bm2):
    n = x.shape[0]
    e = edge_index.shape[1]
    nrows = e // 128
    row2d = edge_index[0].reshape(nrows, 128)
    col2d = edge_index[1].reshape(nrows, 128)
    attr2d = edge_attr.reshape(nrows, 128)

    cnt2 = _make_deg_kernel(n, nrows)(col2d)

    B = 2000
    g = n // B
    dis16, x16 = pl.pallas_call(
        _tc_dis_kernel,
        grid=(g,),
        in_specs=[
            pl.BlockSpec((B, 16), lambda i: (i, 0)),
            pl.BlockSpec((B, 16), lambda i, gg=g: (i + gg, 0)),
            pl.BlockSpec((B, 2), lambda i: (i, 0)),
        ],
        out_specs=[
            pl.BlockSpec((B, 16), lambda i: (i, 0)),
            pl.BlockSpec((B, 16), lambda i: (i, 0)),
        ],
        out_shape=[
            jax.ShapeDtypeStruct((n, 16), jnp.float32),
            jax.ShapeDtypeStruct((n, 16), jnp.float32),
        ],
    )(cnt2, cnt2, x)

    acc1 = _make_agg1_kernel(n, nrows)(row2d, col2d, x16)

    z, zsplit = pl.pallas_call(
        _tc_layer1_kernel,
        grid=(g,),
        in_specs=[
            pl.BlockSpec((B, 16), lambda i: (i, 0)),
            pl.BlockSpec((B, 16), lambda i, gg=g: (i + gg, 0)),
            pl.BlockSpec((B, 16), lambda i: (i, 0)),
            pl.BlockSpec((B, 2), lambda i: (i, 0)),
            pl.BlockSpec((2, 64), lambda i: (0, 0)),
            pl.BlockSpec((1, 64), lambda i: (0, 0)),
            pl.BlockSpec((64, 32), lambda i: (0, 0)),
        ],
        out_specs=[
            pl.BlockSpec((B, 32), lambda i: (i, 0)),
            pl.BlockSpec((2, B, 16), lambda i: (0, i, 0)),
        ],
        out_shape=[
            jax.ShapeDtypeStruct((n, 32), jnp.float32),
            jax.ShapeDtypeStruct((2, n, 16), jnp.float32),
        ],
    )(acc1, acc1, dis16, x, W1, b1.reshape(1, 64), W2)
    zsplit = zsplit.reshape(2 * n, 16)

    acc2 = _make_agg2_kernel(n, nrows)(row2d, col2d, zsplit)

    pq = pl.pallas_call(
        _tc_layer2_kernel,
        grid=(g,),
        in_specs=[
            pl.BlockSpec((B, 16), lambda i: (i, 0)),
            pl.BlockSpec((B, 16), lambda i, gg=g: (i + gg, 0)),
            pl.BlockSpec((B, 16), lambda i: (i, 0)),
            pl.BlockSpec((B, 32), lambda i: (i, 0)),
            pl.BlockSpec((1, 32), lambda i: (0, 0)),
            pl.BlockSpec((32, 16), lambda i: (0, 0)),
            pl.BlockSpec((32, 16), lambda i: (0, 0)),
            pl.BlockSpec((1, 16), lambda i: (0, 0)),
        ],
        out_specs=pl.BlockSpec((2, B, 16), lambda i: (0, i, 0)),
        out_shape=jax.ShapeDtypeStruct((2, n, 16), jnp.float32),
    )(acc2, acc2, dis16, z, b2.reshape(1, 32), Wm1[:32], Wm1[32:64],
      bm1.reshape(1, 16))
    pq = pq.reshape(2 * n, 16)

    cmat = jnp.broadcast_to(Wm1[64][:, None], (16, 16))
    w2mat = jnp.broadcast_to(Wm2[:, 0][:, None], (16, 16))
    bm2v = jnp.full((16,), bm2[0], jnp.float32)
    out2d = _make_mlp_kernel(n, nrows)(
        row2d, col2d, attr2d, pq, cmat, w2mat, bm2v)
    return out2d.reshape(e, 1)
